# baseline (speedup 1.0000x reference)
import jax
import jax.numpy as jnp
from jax.experimental import pallas as pl

_S = 16
_H = 50
_W = 50
_N = _H * _W
_K = 768
_O = 512
_BLK_N = 512


def _mm_kernel(w_ref, p_ref, b_ref, o_ref):
    o_ref[...] = (
        jax.lax.dot_general(
            w_ref[...], p_ref[...],
            (((1,), (1,)), ((), ())),
            preferred_element_type=jnp.float32,
        )
        + b_ref[...]
    )


def kernel(x, W_ext, b_ext, W_conv1, b_conv1, W_loc, b_loc, W_score, b_score):
    patches = (
        x[0]
        .reshape(3, _H, _S, _W, _S)
        .transpose(1, 3, 0, 2, 4)
        .reshape(_N, _K)
        .astype(jnp.bfloat16)
    )
    w_flat = W_ext.reshape(_O, _K).astype(jnp.bfloat16)
    bias = b_ext.reshape(_O, 1)

    out = pl.pallas_call(
        _mm_kernel,
        grid=(pl.cdiv(_N, _BLK_N),),
        in_specs=[
            pl.BlockSpec((_O, _K), lambda n: (0, 0)),
            pl.BlockSpec((_BLK_N, _K), lambda n: (n, 0)),
            pl.BlockSpec((_O, 1), lambda n: (0, 0)),
        ],
        out_specs=pl.BlockSpec((_O, _BLK_N), lambda n: (0, n)),
        out_shape=jax.ShapeDtypeStruct((_O, _N), jnp.float32),
    )(w_flat, patches, bias)

    return out.reshape(1, _O, _H, _W)
